# Initial kernel scaffold; baseline (speedup 1.0000x reference)
import jax
import jax.numpy as jnp
from jax.experimental import pallas as pl


def kernel(x, edge_index, batch, params):
    raise NotImplementedError("write your pallas kernel here")


# speedup vs baseline: 19.0959x; 19.0959x over previous
import functools
import jax
import jax.numpy as jnp
from jax import lax
from jax.experimental import pallas as pl
from jax.experimental.pallas import tpu as pltpu
from jax.experimental.pallas import tpu_sc as plsc

N = 50000
RB = 128
TN = 50048
GRID = TN // RB
NUM_IDS = 2048
IN_CH = 12
HID = 32
LAT = 32
F32 = jnp.float32
NWORK = 32
ECHUNK = 128


def _sc_gat(src, dst, htab, asT, adT, H, C):
    SUB = TN // 16
    ZC = 136
    NZ = SUB // ZC
    ZD = 3136
    EP = src.shape[0]
    EPW = EP // NWORK
    CHUNKS = EPW // ECHUNK
    mesh = plsc.VectorSubcoreMesh(core_axis_name="c", subcore_axis_name="s")

    @functools.partial(
        pl.kernel,
        out_type=(jax.ShapeDtypeStruct((2, H, TN, C), F32),
                  jax.ShapeDtypeStruct((H * 2 * TN,), F32)),
        mesh=mesh,
        compiler_params=pltpu.CompilerParams(use_tc_tiling_on_sc=False),
        scratch_types=[
            pltpu.VMEM((ECHUNK,), jnp.int32),
            pltpu.VMEM((ECHUNK,), jnp.int32),
            pltpu.VMEM((ECHUNK,), jnp.int32),
            pltpu.VMEM((ECHUNK,), jnp.int32),
            pltpu.VMEM((ECHUNK,), F32),
            pltpu.VMEM((ECHUNK,), F32),
            pltpu.VMEM((ECHUNK,), F32),
            pltpu.VMEM((ECHUNK, C), F32),
            pltpu.VMEM((ECHUNK, C), F32),
            pltpu.VMEM((ZC, C), F32),
            pltpu.VMEM((ZD,), F32),
            pltpu.VMEM_SHARED((TN, C), F32),
            pltpu.VMEM_SHARED((TN,), F32),
        ],
    )
    def k(src_hbm, dst_hbm, htab_hbm, asT_hbm, adT_hbm, num_hbm, den_hbm,
          src_v, dst_v, idxs_v, idxd_v, as_v, ad_v, w_v, rows_v, val_v,
          zero_v, zd_v, acc_n, acc_d):
        cid = lax.axis_index("c")
        sid = lax.axis_index("s")
        wbase = (cid * 16 + sid) * EPW
        zrow = sid * SUB
        z16 = jnp.zeros((16,), F32)

        def zfill(r, carry):
            for c0 in range(0, C, 16):
                zero_v[r, pl.ds(c0, 16)] = z16
            return carry
        lax.fori_loop(0, ZC, zfill, 0)

        def zdfill(g, carry):
            zd_v[pl.ds(g * 16, 16)] = z16
            return carry
        lax.fori_loop(0, ZD // 16, zdfill, 0)

        for h in range(H):
            def zero_body(j, carry):
                pltpu.sync_copy(zero_v, acc_n.at[pl.ds(zrow + j * ZC, ZC)])
                return carry
            lax.fori_loop(0, NZ, zero_body, 0)
            pltpu.sync_copy(zd_v.at[pl.ds(0, SUB)], acc_d.at[pl.ds(zrow, SUB)])
            plsc.subcore_barrier()

            def chunk_body(i, carry):
                base = wbase + i * ECHUNK
                pltpu.sync_copy(src_hbm.at[pl.ds(base, ECHUNK)], src_v)
                pltpu.sync_copy(dst_hbm.at[pl.ds(base, ECHUNK)], dst_v)
                if h == 0:
                    s_idx, d_idx = src_v, dst_v
                else:
                    def addoff(g, c2):
                        sl = pl.ds(g * 16, 16)
                        idxs_v[sl] = src_v[sl] + (h * TN)
                        idxd_v[sl] = dst_v[sl] + (h * TN)
                        return c2
                    lax.fori_loop(0, ECHUNK // 16, addoff, 0)
                    s_idx, d_idx = idxs_v, idxd_v
                pltpu.sync_copy(asT_hbm.at[s_idx], as_v)
                pltpu.sync_copy(adT_hbm.at[d_idx], ad_v)
                pltpu.sync_copy(htab_hbm.at[h].at[src_v], rows_v)

                def gbody(g, c2):
                    sl = pl.ds(g * 16, 16)
                    a16 = as_v[sl] + ad_v[sl]
                    a16 = jnp.where(a16 > 0, a16, a16 * 0.2)
                    w16 = jnp.exp(a16)
                    w_v[sl] = w16
                    for j in range(16):
                        e = g * 16 + j
                        wv = w16[j]
                        for c0 in range(0, C, 16):
                            csl = pl.ds(c0, 16)
                            val_v[e, csl] = rows_v[e, csl] * wv
                    return c2
                lax.fori_loop(0, ECHUNK // 16, gbody, 0)

                pltpu.sync_copy(val_v, acc_n.at[dst_v], add=True)
                pltpu.sync_copy(w_v, acc_d.at[dst_v], add=True)
                return carry
            lax.fori_loop(0, CHUNKS, chunk_body, 0)
            plsc.subcore_barrier()

            pltpu.sync_copy(acc_n.at[pl.ds(zrow, SUB)],
                            num_hbm.at[cid, h, pl.ds(zrow, SUB)])
            pltpu.sync_copy(
                acc_d.at[pl.ds(zrow, SUB)],
                den_hbm.at[pl.ds(h * 2 * TN + cid * TN + zrow, SUB)])
            plsc.subcore_barrier()

    num, den = k(src, dst, htab, asT.reshape(-1), adT.reshape(-1))
    return num, den.reshape(H, 2, TN)


def _full(shape):
    nd = len(shape)
    return pl.BlockSpec(shape, lambda i: (0,) * nd)


def _pre1(x, emb, W, a_s, a_d):
    def body(x_ref, emb_ref, W_ref, as_ref, ad_ref, ht_ref, asT_ref, adT_ref):
        xb = x_ref[...]
        ids = xb[:, 0].astype(jnp.int32)
        oh = (lax.broadcasted_iota(jnp.int32, (RB, NUM_IDS), 1)
              == ids[:, None]).astype(F32)
        e = jnp.dot(oh, emb_ref[...], preferred_element_type=F32)
        u = jnp.concatenate([e, xb[:, 1:]], axis=1)
        for h in range(4):
            hh = jnp.dot(u, W_ref[:, h * 32:(h + 1) * 32],
                         preferred_element_type=F32)
            ht_ref[h] = hh
            asT_ref[h] = jnp.sum(hh * as_ref[h][None, :], axis=1)
            adT_ref[h] = jnp.sum(hh * ad_ref[h][None, :], axis=1)

    return pl.pallas_call(
        body,
        grid=(GRID,),
        in_specs=[
            pl.BlockSpec((RB, IN_CH), lambda i: (i, 0)),
            _full((NUM_IDS, 8)),
            _full((19, 128)),
            _full((8, 32)),
            _full((8, 32)),
        ],
        out_specs=[
            pl.BlockSpec((4, RB, 32), lambda i: (0, i, 0)),
            pl.BlockSpec((8, RB), lambda i: (0, i)),
            pl.BlockSpec((8, RB), lambda i: (0, i)),
        ],
        out_shape=[
            jax.ShapeDtypeStruct((4, TN, 32), F32),
            jax.ShapeDtypeStruct((8, TN), F32),
            jax.ShapeDtypeStruct((8, TN), F32),
        ],
    )(x, emb, W, a_s, a_d)


def _pre(u, stats, bn, W, a_s, a_d, H, C, Fin, emit_y):
    with_bn = bn is not None

    def body(*refs):
        idx = 0
        u_ref = refs[idx]; idx += 1
        if with_bn:
            s1_ref = refs[idx]; s2_ref = refs[idx + 1]
            g_ref = refs[idx + 2]; be_ref = refs[idx + 3]
            idx += 4
        W_ref = refs[idx]; as_ref = refs[idx + 1]; ad_ref = refs[idx + 2]
        idx += 3
        ht_ref = refs[idx]; asT_ref = refs[idx + 1]; adT_ref = refs[idx + 2]
        idx += 3
        ub = u_ref[...]
        if with_bn:
            m = s1_ref[0] * (1.0 / N)
            v = s2_ref[0] * (1.0 / N) - m * m
            ub = (g_ref[0][None, :] * (ub - m[None, :])
                  * lax.rsqrt(v + 1e-5)[None, :] + be_ref[0][None, :])
            ub = jnp.maximum(ub, 0.0)
        if emit_y:
            refs[idx][...] = ub
        for h in range(H):
            hh = jnp.dot(ub, W_ref[:, h * C:(h + 1) * C],
                         preferred_element_type=F32)
            ht_ref[h] = hh
            asT_ref[h] = jnp.sum(hh * as_ref[h][None, :], axis=1)
            adT_ref[h] = jnp.sum(hh * ad_ref[h][None, :], axis=1)

    in_specs = [pl.BlockSpec((RB, Fin), lambda i: (i, 0))]
    args = [u]
    if with_bn:
        s1, s2 = stats
        in_specs += [_full((1, Fin))] * 4
        args += [s1, s2, bn["g"].reshape(1, Fin), bn["be"].reshape(1, Fin)]
    in_specs += [_full((Fin, H * C)), _full((8, C)), _full((8, C))]
    args += [W, a_s, a_d]
    out_specs = [
        pl.BlockSpec((H, RB, C), lambda i: (0, i, 0)),
        pl.BlockSpec((8, RB), lambda i: (0, i)),
        pl.BlockSpec((8, RB), lambda i: (0, i)),
    ]
    out_shape = [
        jax.ShapeDtypeStruct((H, TN, C), F32),
        jax.ShapeDtypeStruct((8, TN), F32),
        jax.ShapeDtypeStruct((8, TN), F32),
    ]
    if emit_y:
        out_specs.append(pl.BlockSpec((RB, Fin), lambda i: (i, 0)))
        out_shape.append(jax.ShapeDtypeStruct((TN, Fin), F32))
    return pl.pallas_call(
        body, grid=(GRID,), in_specs=in_specs,
        out_specs=out_specs, out_shape=out_shape,
    )(*args)


def _post(sc, b, H, C):
    F = H * C
    scn, scd = sc

    def body(sc_ref, dn_ref, b_ref, raw_ref, s1_ref, s2_ref):
        so = sc_ref[...]
        dn = dn_ref[...]
        num = so[0] + so[1]
        den = dn[:, 0] + dn[:, 1]
        outp = num / (den[:, :, None] + 1e-16)
        raw = jnp.concatenate([outp[h] for h in range(H)], axis=1) + b_ref[0]
        raw_ref[...] = raw
        i = pl.program_id(0)
        mask = (lax.broadcasted_iota(jnp.int32, (RB, 1), 0) + i * RB) < N
        rm = jnp.where(mask, raw, 0.0)
        p1 = jnp.sum(rm, axis=0, keepdims=True)
        p2 = jnp.sum(rm * rm, axis=0, keepdims=True)

        @pl.when(i == 0)
        def _():
            s1_ref[...] = p1
            s2_ref[...] = p2

        @pl.when(i != 0)
        def _():
            s1_ref[...] += p1
            s2_ref[...] += p2

    return pl.pallas_call(
        body,
        grid=(GRID,),
        in_specs=[
            pl.BlockSpec((2, H, RB, C), lambda i: (0, 0, i, 0)),
            pl.BlockSpec((H, 2, RB), lambda i: (0, 0, i)),
            _full((1, F)),
        ],
        out_specs=[
            pl.BlockSpec((RB, F), lambda i: (i, 0)),
            pl.BlockSpec((1, F), lambda i: (0, 0)),
            pl.BlockSpec((1, F), lambda i: (0, 0)),
        ],
        out_shape=[
            jax.ShapeDtypeStruct((TN, F), F32),
            jax.ShapeDtypeStruct((1, F), F32),
            jax.ShapeDtypeStruct((1, F), F32),
        ],
    )(scn, scd, b.reshape(1, F))


def _zk(raw3, s1, s2, bn, y2, zm, zl, eps):
    def body(r_ref, s1_ref, s2_ref, g_ref, be_ref, y2_ref, zmW_ref, zmb_ref,
             zlW_ref, zlb_ref, eps_ref, z_ref, klp_ref):
        m = s1_ref[0] * (1.0 / N)
        v = s2_ref[0] * (1.0 / N) - m * m
        x3 = (g_ref[0][None, :] * (r_ref[...] - m[None, :])
              * lax.rsqrt(v + 1e-5)[None, :] + be_ref[0][None, :])
        x3 = jnp.maximum(x3, 0.0)
        hh = x3 + y2_ref[...]
        mu = jnp.dot(hh, zmW_ref[...], preferred_element_type=F32) + zmb_ref[0]
        lv = jnp.dot(hh, zlW_ref[...], preferred_element_type=F32) + zlb_ref[0]
        z = mu + eps_ref[...] * jnp.exp(0.5 * lv)
        z_ref[...] = z
        i = pl.program_id(0)
        mask = (lax.broadcasted_iota(jnp.int32, (RB, 1), 0) + i * RB) < N
        t = jnp.where(mask, 1.0 + lv - mu * mu - jnp.exp(lv), 0.0)
        p = jnp.sum(t).reshape(1, 1)

        @pl.when(i == 0)
        def _():
            klp_ref[...] = p

        @pl.when(i != 0)
        def _():
            klp_ref[...] += p

    return pl.pallas_call(
        body,
        grid=(GRID,),
        in_specs=[
            pl.BlockSpec((RB, HID), lambda i: (i, 0)),
            _full((1, HID)), _full((1, HID)), _full((1, HID)), _full((1, HID)),
            pl.BlockSpec((RB, HID), lambda i: (i, 0)),
            _full((HID, LAT)), _full((1, LAT)),
            _full((HID, LAT)), _full((1, LAT)),
            pl.BlockSpec((RB, LAT), lambda i: (i, 0)),
        ],
        out_specs=[
            pl.BlockSpec((RB, LAT), lambda i: (i, 0)),
            pl.BlockSpec((1, 1), lambda i: (0, 0)),
        ],
        out_shape=[
            jax.ShapeDtypeStruct((TN, LAT), F32),
            jax.ShapeDtypeStruct((1, 1), F32),
        ],
    )(raw3, s1, s2, bn["g"].reshape(1, HID), bn["be"].reshape(1, HID),
      y2, zm["W"], zm["b"].reshape(1, LAT), zl["W"], zl["b"].reshape(1, LAT),
      eps)


def _final(sc, b, d2, cW, cb):
    scn, scd = sc

    def body(sc_ref, dn_ref, b_ref, d2_ref, cW_ref, cb_ref, cont_ref, can_ref):
        so = sc_ref[...]
        dn = dn_ref[...]
        num = so[0, 0] + so[1, 0]
        den = dn[0, 0] + dn[0, 1]
        raw = num / (den[:, None] + 1e-16) + b_ref[0]
        cont_ref[...] = 1.0 / (1.0 + jnp.exp(-raw))
        can_ref[...] = (jnp.dot(d2_ref[...], cW_ref[...],
                                preferred_element_type=F32) + cb_ref[0])

    return pl.pallas_call(
        body,
        grid=(GRID,),
        in_specs=[
            pl.BlockSpec((2, 1, RB, 16), lambda i: (0, 0, i, 0)),
            pl.BlockSpec((1, 2, RB), lambda i: (0, 0, i)),
            _full((1, 16)),
            pl.BlockSpec((RB, HID), lambda i: (i, 0)),
            _full((HID, NUM_IDS)),
            _full((1, NUM_IDS)),
        ],
        out_specs=[
            pl.BlockSpec((RB, 16), lambda i: (i, 0)),
            pl.BlockSpec((RB, NUM_IDS), lambda i: (i, 0)),
        ],
        out_shape=[
            jax.ShapeDtypeStruct((TN, 16), F32),
            jax.ShapeDtypeStruct((TN, NUM_IDS), F32),
        ],
    )(scn, scd, b.reshape(1, 16), d2, cW, cb.reshape(1, NUM_IDS))


def _att(pr, H, C, Cpad=None):
    Cp = Cpad or C
    a_s = jnp.pad(pr["att_src"].reshape(H, C), ((0, 8 - H), (0, Cp - C)))
    a_d = jnp.pad(pr["att_dst"].reshape(H, C), ((0, 8 - H), (0, Cp - C)))
    return a_s, a_d


def kernel(x, edge_index, batch, params):
    p = params
    xp = jnp.pad(x, ((0, TN - N), (0, 0)))
    loop = jnp.arange(N, dtype=edge_index.dtype)
    src = jnp.concatenate([edge_index[0], loop])
    dst = jnp.concatenate([edge_index[1], loop])
    EE = src.shape[0]
    EP = ((EE + NWORK * ECHUNK - 1) // (NWORK * ECHUNK)) * (NWORK * ECHUNK)
    src = jnp.pad(src, (0, EP - EE))
    dst = jnp.pad(dst, (0, EP - EE), constant_values=N)
    eps = jax.random.normal(jax.random.key(1), (N, LAT), F32)
    epsp = jnp.pad(eps, ((0, TN - N), (0, 0)))

    h1, as1, ad1 = _pre1(xp, p["emb"], p["enc1"]["W"], *_att(p["enc1"], 4, 32))
    sc1 = _sc_gat(src, dst, h1, as1, ad1, 4, 32)
    raw1, s11, s21 = _post(sc1, p["enc1"]["b"], 4, 32)

    h2, as2, ad2 = _pre(raw1, (s11, s21), p["bn1"], p["enc2"]["W"],
                        *_att(p["enc2"], 1, 32), 1, 32, 128, False)
    sc2 = _sc_gat(src, dst, h2, as2, ad2, 1, 32)
    raw2, s12, s22 = _post(sc2, p["enc2"]["b"], 1, 32)

    h3, as3, ad3, y2 = _pre(raw2, (s12, s22), p["bn2"], p["enc3"]["W"],
                            *_att(p["enc3"], 1, 32), 1, 32, 32, True)
    sc3 = _sc_gat(src, dst, h3, as3, ad3, 1, 32)
    raw3, s13, s23 = _post(sc3, p["enc3"]["b"], 1, 32)

    zp, klp = _zk(raw3, s13, s23, p["bn3"], y2, p["zm"], p["zl"], epsp)

    hd1, asd1, add1 = _pre(zp, None, None, p["dec1"]["W"],
                           *_att(p["dec1"], 4, 32), 4, 32, LAT, False)
    scd1 = _sc_gat(src, dst, hd1, asd1, add1, 4, 32)
    rawd1, sd11, sd21 = _post(scd1, p["dec1"]["b"], 4, 32)

    hd2, asd2, add2 = _pre(rawd1, (sd11, sd21), p["dbn1"], p["dec2"]["W"],
                           *_att(p["dec2"], 1, 32), 1, 32, 128, False)
    scd2 = _sc_gat(src, dst, hd2, asd2, add2, 1, 32)
    rawd2, sd12, sd22 = _post(scd2, p["dec2"]["b"], 1, 32)

    Wd3 = jnp.pad(p["dec3"]["W"], ((0, 0), (0, 5)))
    hd3, asd3, add3, d2 = _pre(rawd2, (sd12, sd22), p["dbn2"], Wd3,
                               *_att(p["dec3"], 1, 11, 16), 1, 16, 32, True)
    scd3 = _sc_gat(src, dst, hd3, asd3, add3, 1, 16)
    bd3 = jnp.pad(p["dec3"]["b"], (0, 5))
    contp, canp = _final(scd3, bd3, d2, p["cls"]["W"], p["cls"]["b"])

    cont = contp[:N, :11]
    canid = canp[:N]
    z = zp[:N]
    kl = -0.5 * klp[0, 0] / (N * LAT)
    return cont, canid, z, kl


# speedup vs baseline: 30.8436x; 1.6152x over previous
import functools
import jax
import jax.numpy as jnp
from jax import lax
from jax.experimental import pallas as pl
from jax.experimental.pallas import tpu as pltpu
from jax.experimental.pallas import tpu_sc as plsc

N = 50000
RB = 128
TN = 50048
GRID = TN // RB
NUM_IDS = 2048
IN_CH = 12
HID = 32
LAT = 32
F32 = jnp.float32
NWORK = 32
ECHUNK = 128


def _sc_gat(src, dst, htab, asT, adT, H, C):
    SUB = TN // 16
    ZC = 136
    NZ = SUB // ZC
    ZD = 3136
    EP = src.shape[0]
    EPW = EP // NWORK
    CHUNKS = EPW // ECHUNK
    mesh = plsc.VectorSubcoreMesh(core_axis_name="c", subcore_axis_name="s")

    @functools.partial(
        pl.kernel,
        out_type=(jax.ShapeDtypeStruct((2, H, TN, C), F32),
                  jax.ShapeDtypeStruct((H * 2 * TN,), F32)),
        mesh=mesh,
        compiler_params=pltpu.CompilerParams(use_tc_tiling_on_sc=False),
        scratch_types=[
            pltpu.VMEM((2, ECHUNK), jnp.int32),
            pltpu.VMEM((2, ECHUNK), jnp.int32),
            pltpu.VMEM((2, ECHUNK), jnp.int32),
            pltpu.VMEM((2, ECHUNK), jnp.int32),
            pltpu.VMEM((2, ECHUNK), F32),
            pltpu.VMEM((2, ECHUNK), F32),
            pltpu.VMEM((2, ECHUNK), F32),
            pltpu.VMEM((2, ECHUNK, C), F32),
            pltpu.VMEM((2, ECHUNK, C), F32),
            pltpu.VMEM((ZC, C), F32),
            pltpu.VMEM((ZD,), F32),
            pltpu.VMEM_SHARED((TN, C), F32),
            pltpu.VMEM_SHARED((TN,), F32),
            pltpu.SemaphoreType.DMA,
            pltpu.SemaphoreType.DMA,
        ],
    )
    def k(src_hbm, dst_hbm, htab_hbm, asT_hbm, adT_hbm, num_hbm, den_hbm,
          src_v, dst_v, idxs_v, idxd_v, as_v, ad_v, w_v, rows_v, val_v,
          zero_v, zd_v, acc_n, acc_d, sem0, sem1):
        cid = lax.axis_index("c")
        sid = lax.axis_index("s")
        wbase = (cid * 16 + sid) * EPW
        zrow = sid * SUB
        z16 = jnp.zeros((16,), F32)

        def zfill(r, carry):
            for c0 in range(0, C, 16):
                zero_v[r, pl.ds(c0, 16)] = z16
            return carry
        lax.fori_loop(0, ZC, zfill, 0)

        def zdfill(g, carry):
            zd_v[pl.ds(g * 16, 16)] = z16
            return carry
        lax.fori_loop(0, ZD // 16, zdfill, 0)

        for h in range(H):
            def zero_body(j, carry):
                pltpu.sync_copy(zero_v, acc_n.at[pl.ds(zrow + j * ZC, ZC)])
                return carry
            lax.fori_loop(0, NZ, zero_body, 0)
            pltpu.sync_copy(zd_v.at[pl.ds(0, SUB)], acc_d.at[pl.ds(zrow, SUB)])
            plsc.subcore_barrier()

            sems = (sem0, sem1)

            def load_and_issue(i, b):
                base = wbase + i * ECHUNK
                pltpu.sync_copy(src_hbm.at[pl.ds(base, ECHUNK)], src_v.at[b])
                pltpu.sync_copy(dst_hbm.at[pl.ds(base, ECHUNK)], dst_v.at[b])
                if h == 0:
                    s_idx, d_idx = src_v.at[b], dst_v.at[b]
                else:
                    def addoff(g, c2):
                        sl = pl.ds(g * 16, 16)
                        idxs_v[b, sl] = src_v[b, sl] + (h * TN)
                        idxd_v[b, sl] = dst_v[b, sl] + (h * TN)
                        return c2
                    lax.fori_loop(0, ECHUNK // 16, addoff, 0)
                    s_idx, d_idx = idxs_v.at[b], idxd_v.at[b]
                pltpu.async_copy(asT_hbm.at[s_idx], as_v.at[b], sems[b])
                pltpu.async_copy(adT_hbm.at[d_idx], ad_v.at[b], sems[b])
                pltpu.async_copy(htab_hbm.at[h].at[src_v.at[b]],
                                 rows_v.at[b], sems[b])

            def wait_gathers(b):
                pltpu.make_async_copy(asT_hbm.at[pl.ds(0, ECHUNK)],
                                      as_v.at[b], sems[b]).wait()
                pltpu.make_async_copy(adT_hbm.at[pl.ds(0, ECHUNK)],
                                      ad_v.at[b], sems[b]).wait()
                pltpu.make_async_copy(htab_hbm.at[h].at[pl.ds(0, ECHUNK)],
                                      rows_v.at[b], sems[b]).wait()

            def compute_scatter(b):
                def gbody(g, c2):
                    sl = pl.ds(g * 16, 16)
                    a16 = as_v[b, sl] + ad_v[b, sl]
                    a16 = jnp.where(a16 > 0, a16, a16 * 0.2)
                    w16 = jnp.exp(a16)
                    w_v[b, sl] = w16
                    for j in range(16):
                        e = g * 16 + j
                        wv = w16[j]
                        for c0 in range(0, C, 16):
                            csl = pl.ds(c0, 16)
                            val_v[b, e, csl] = rows_v[b, e, csl] * wv
                    return c2
                lax.fori_loop(0, ECHUNK // 16, gbody, 0)
                pltpu.sync_copy(val_v.at[b], acc_n.at[dst_v.at[b]], add=True)
                pltpu.sync_copy(w_v.at[b], acc_d.at[dst_v.at[b]], add=True)

            load_and_issue(0, 0)

            def dchunk(i2, carry):
                for b in range(2):
                    i = i2 * 2 + b
                    nb = 1 - b

                    @pl.when(i + 1 < CHUNKS)
                    def _():
                        load_and_issue(i + 1, nb)
                    wait_gathers(b)
                    compute_scatter(b)
                return carry
            lax.fori_loop(0, CHUNKS // 2, dchunk, 0)
            plsc.subcore_barrier()

            pltpu.sync_copy(acc_n.at[pl.ds(zrow, SUB)],
                            num_hbm.at[cid, h, pl.ds(zrow, SUB)])
            pltpu.sync_copy(
                acc_d.at[pl.ds(zrow, SUB)],
                den_hbm.at[pl.ds(h * 2 * TN + cid * TN + zrow, SUB)])
            plsc.subcore_barrier()

    num, den = k(src, dst, htab, asT.reshape(-1), adT.reshape(-1))
    return num, den.reshape(H, 2, TN)


def _full(shape):
    nd = len(shape)
    return pl.BlockSpec(shape, lambda i: (0,) * nd)


def _pre1(x, emb, W, a_s, a_d):
    def body(x_ref, emb_ref, W_ref, as_ref, ad_ref, ht_ref, asT_ref, adT_ref):
        xb = x_ref[...]
        ids = xb[:, 0].astype(jnp.int32)
        oh = (lax.broadcasted_iota(jnp.int32, (RB, NUM_IDS), 1)
              == ids[:, None]).astype(F32)
        e = jnp.dot(oh, emb_ref[...], preferred_element_type=F32)
        u = jnp.concatenate([e, xb[:, 1:]], axis=1)
        for h in range(4):
            hh = jnp.dot(u, W_ref[:, h * 32:(h + 1) * 32],
                         preferred_element_type=F32)
            ht_ref[h] = hh
            asT_ref[h] = jnp.sum(hh * as_ref[h][None, :], axis=1)
            adT_ref[h] = jnp.sum(hh * ad_ref[h][None, :], axis=1)

    return pl.pallas_call(
        body,
        grid=(GRID,),
        in_specs=[
            pl.BlockSpec((RB, IN_CH), lambda i: (i, 0)),
            _full((NUM_IDS, 8)),
            _full((19, 128)),
            _full((8, 32)),
            _full((8, 32)),
        ],
        out_specs=[
            pl.BlockSpec((4, RB, 32), lambda i: (0, i, 0)),
            pl.BlockSpec((8, RB), lambda i: (0, i)),
            pl.BlockSpec((8, RB), lambda i: (0, i)),
        ],
        out_shape=[
            jax.ShapeDtypeStruct((4, TN, 32), F32),
            jax.ShapeDtypeStruct((8, TN), F32),
            jax.ShapeDtypeStruct((8, TN), F32),
        ],
    )(x, emb, W, a_s, a_d)


def _pre(u, stats, bn, W, a_s, a_d, H, C, Fin, emit_y):
    with_bn = bn is not None

    def body(*refs):
        idx = 0
        u_ref = refs[idx]; idx += 1
        if with_bn:
            s1_ref = refs[idx]; s2_ref = refs[idx + 1]
            g_ref = refs[idx + 2]; be_ref = refs[idx + 3]
            idx += 4
        W_ref = refs[idx]; as_ref = refs[idx + 1]; ad_ref = refs[idx + 2]
        idx += 3
        ht_ref = refs[idx]; asT_ref = refs[idx + 1]; adT_ref = refs[idx + 2]
        idx += 3
        ub = u_ref[...]
        if with_bn:
            m = s1_ref[0] * (1.0 / N)
            v = s2_ref[0] * (1.0 / N) - m * m
            ub = (g_ref[0][None, :] * (ub - m[None, :])
                  * lax.rsqrt(v + 1e-5)[None, :] + be_ref[0][None, :])
            ub = jnp.maximum(ub, 0.0)
        if emit_y:
            refs[idx][...] = ub
        for h in range(H):
            hh = jnp.dot(ub, W_ref[:, h * C:(h + 1) * C],
                         preferred_element_type=F32)
            ht_ref[h] = hh
            asT_ref[h] = jnp.sum(hh * as_ref[h][None, :], axis=1)
            adT_ref[h] = jnp.sum(hh * ad_ref[h][None, :], axis=1)

    in_specs = [pl.BlockSpec((RB, Fin), lambda i: (i, 0))]
    args = [u]
    if with_bn:
        s1, s2 = stats
        in_specs += [_full((1, Fin))] * 4
        args += [s1, s2, bn["g"].reshape(1, Fin), bn["be"].reshape(1, Fin)]
    in_specs += [_full((Fin, H * C)), _full((8, C)), _full((8, C))]
    args += [W, a_s, a_d]
    out_specs = [
        pl.BlockSpec((H, RB, C), lambda i: (0, i, 0)),
        pl.BlockSpec((8, RB), lambda i: (0, i)),
        pl.BlockSpec((8, RB), lambda i: (0, i)),
    ]
    out_shape = [
        jax.ShapeDtypeStruct((H, TN, C), F32),
        jax.ShapeDtypeStruct((8, TN), F32),
        jax.ShapeDtypeStruct((8, TN), F32),
    ]
    if emit_y:
        out_specs.append(pl.BlockSpec((RB, Fin), lambda i: (i, 0)))
        out_shape.append(jax.ShapeDtypeStruct((TN, Fin), F32))
    return pl.pallas_call(
        body, grid=(GRID,), in_specs=in_specs,
        out_specs=out_specs, out_shape=out_shape,
    )(*args)


def _post(sc, b, H, C):
    F = H * C
    scn, scd = sc

    def body(sc_ref, dn_ref, b_ref, raw_ref, s1_ref, s2_ref):
        so = sc_ref[...]
        dn = dn_ref[...]
        num = so[0] + so[1]
        den = dn[:, 0] + dn[:, 1]
        outp = num / (den[:, :, None] + 1e-16)
        raw = jnp.concatenate([outp[h] for h in range(H)], axis=1) + b_ref[0]
        raw_ref[...] = raw
        i = pl.program_id(0)
        mask = (lax.broadcasted_iota(jnp.int32, (RB, 1), 0) + i * RB) < N
        rm = jnp.where(mask, raw, 0.0)
        p1 = jnp.sum(rm, axis=0, keepdims=True)
        p2 = jnp.sum(rm * rm, axis=0, keepdims=True)

        @pl.when(i == 0)
        def _():
            s1_ref[...] = p1
            s2_ref[...] = p2

        @pl.when(i != 0)
        def _():
            s1_ref[...] += p1
            s2_ref[...] += p2

    return pl.pallas_call(
        body,
        grid=(GRID,),
        in_specs=[
            pl.BlockSpec((2, H, RB, C), lambda i: (0, 0, i, 0)),
            pl.BlockSpec((H, 2, RB), lambda i: (0, 0, i)),
            _full((1, F)),
        ],
        out_specs=[
            pl.BlockSpec((RB, F), lambda i: (i, 0)),
            pl.BlockSpec((1, F), lambda i: (0, 0)),
            pl.BlockSpec((1, F), lambda i: (0, 0)),
        ],
        out_shape=[
            jax.ShapeDtypeStruct((TN, F), F32),
            jax.ShapeDtypeStruct((1, F), F32),
            jax.ShapeDtypeStruct((1, F), F32),
        ],
    )(scn, scd, b.reshape(1, F))


def _zk(raw3, s1, s2, bn, y2, zm, zl, eps):
    def body(r_ref, s1_ref, s2_ref, g_ref, be_ref, y2_ref, zmW_ref, zmb_ref,
             zlW_ref, zlb_ref, eps_ref, z_ref, klp_ref):
        m = s1_ref[0] * (1.0 / N)
        v = s2_ref[0] * (1.0 / N) - m * m
        x3 = (g_ref[0][None, :] * (r_ref[...] - m[None, :])
              * lax.rsqrt(v + 1e-5)[None, :] + be_ref[0][None, :])
        x3 = jnp.maximum(x3, 0.0)
        hh = x3 + y2_ref[...]
        mu = jnp.dot(hh, zmW_ref[...], preferred_element_type=F32) + zmb_ref[0]
        lv = jnp.dot(hh, zlW_ref[...], preferred_element_type=F32) + zlb_ref[0]
        z = mu + eps_ref[...] * jnp.exp(0.5 * lv)
        z_ref[...] = z
        i = pl.program_id(0)
        mask = (lax.broadcasted_iota(jnp.int32, (RB, 1), 0) + i * RB) < N
        t = jnp.where(mask, 1.0 + lv - mu * mu - jnp.exp(lv), 0.0)
        p = jnp.sum(t).reshape(1, 1)

        @pl.when(i == 0)
        def _():
            klp_ref[...] = p

        @pl.when(i != 0)
        def _():
            klp_ref[...] += p

    return pl.pallas_call(
        body,
        grid=(GRID,),
        in_specs=[
            pl.BlockSpec((RB, HID), lambda i: (i, 0)),
            _full((1, HID)), _full((1, HID)), _full((1, HID)), _full((1, HID)),
            pl.BlockSpec((RB, HID), lambda i: (i, 0)),
            _full((HID, LAT)), _full((1, LAT)),
            _full((HID, LAT)), _full((1, LAT)),
            pl.BlockSpec((RB, LAT), lambda i: (i, 0)),
        ],
        out_specs=[
            pl.BlockSpec((RB, LAT), lambda i: (i, 0)),
            pl.BlockSpec((1, 1), lambda i: (0, 0)),
        ],
        out_shape=[
            jax.ShapeDtypeStruct((TN, LAT), F32),
            jax.ShapeDtypeStruct((1, 1), F32),
        ],
    )(raw3, s1, s2, bn["g"].reshape(1, HID), bn["be"].reshape(1, HID),
      y2, zm["W"], zm["b"].reshape(1, LAT), zl["W"], zl["b"].reshape(1, LAT),
      eps)


def _final(sc, b, d2, cW, cb):
    scn, scd = sc

    def body(sc_ref, dn_ref, b_ref, d2_ref, cW_ref, cb_ref, cont_ref, can_ref):
        so = sc_ref[...]
        dn = dn_ref[...]
        num = so[0, 0] + so[1, 0]
        den = dn[0, 0] + dn[0, 1]
        raw = num / (den[:, None] + 1e-16) + b_ref[0]
        cont_ref[...] = 1.0 / (1.0 + jnp.exp(-raw))
        can_ref[...] = (jnp.dot(d2_ref[...], cW_ref[...],
                                preferred_element_type=F32) + cb_ref[0])

    return pl.pallas_call(
        body,
        grid=(GRID,),
        in_specs=[
            pl.BlockSpec((2, 1, RB, 16), lambda i: (0, 0, i, 0)),
            pl.BlockSpec((1, 2, RB), lambda i: (0, 0, i)),
            _full((1, 16)),
            pl.BlockSpec((RB, HID), lambda i: (i, 0)),
            _full((HID, NUM_IDS)),
            _full((1, NUM_IDS)),
        ],
        out_specs=[
            pl.BlockSpec((RB, 16), lambda i: (i, 0)),
            pl.BlockSpec((RB, NUM_IDS), lambda i: (i, 0)),
        ],
        out_shape=[
            jax.ShapeDtypeStruct((TN, 16), F32),
            jax.ShapeDtypeStruct((TN, NUM_IDS), F32),
        ],
    )(scn, scd, b.reshape(1, 16), d2, cW, cb.reshape(1, NUM_IDS))


def _att(pr, H, C, Cpad=None):
    Cp = Cpad or C
    a_s = jnp.pad(pr["att_src"].reshape(H, C), ((0, 8 - H), (0, Cp - C)))
    a_d = jnp.pad(pr["att_dst"].reshape(H, C), ((0, 8 - H), (0, Cp - C)))
    return a_s, a_d


def kernel(x, edge_index, batch, params):
    p = params
    xp = jnp.pad(x, ((0, TN - N), (0, 0)))
    loop = jnp.arange(N, dtype=edge_index.dtype)
    src = jnp.concatenate([edge_index[0], loop])
    dst = jnp.concatenate([edge_index[1], loop])
    EE = src.shape[0]
    EQ = NWORK * ECHUNK * 2
    EP = ((EE + EQ - 1) // EQ) * EQ
    src = jnp.pad(src, (0, EP - EE))
    dst = jnp.pad(dst, (0, EP - EE), constant_values=N)
    eps = jax.random.normal(jax.random.key(1), (N, LAT), F32)
    epsp = jnp.pad(eps, ((0, TN - N), (0, 0)))

    h1, as1, ad1 = _pre1(xp, p["emb"], p["enc1"]["W"], *_att(p["enc1"], 4, 32))
    sc1 = _sc_gat(src, dst, h1, as1, ad1, 4, 32)
    raw1, s11, s21 = _post(sc1, p["enc1"]["b"], 4, 32)

    h2, as2, ad2 = _pre(raw1, (s11, s21), p["bn1"], p["enc2"]["W"],
                        *_att(p["enc2"], 1, 32), 1, 32, 128, False)
    sc2 = _sc_gat(src, dst, h2, as2, ad2, 1, 32)
    raw2, s12, s22 = _post(sc2, p["enc2"]["b"], 1, 32)

    h3, as3, ad3, y2 = _pre(raw2, (s12, s22), p["bn2"], p["enc3"]["W"],
                            *_att(p["enc3"], 1, 32), 1, 32, 32, True)
    sc3 = _sc_gat(src, dst, h3, as3, ad3, 1, 32)
    raw3, s13, s23 = _post(sc3, p["enc3"]["b"], 1, 32)

    zp, klp = _zk(raw3, s13, s23, p["bn3"], y2, p["zm"], p["zl"], epsp)

    hd1, asd1, add1 = _pre(zp, None, None, p["dec1"]["W"],
                           *_att(p["dec1"], 4, 32), 4, 32, LAT, False)
    scd1 = _sc_gat(src, dst, hd1, asd1, add1, 4, 32)
    rawd1, sd11, sd21 = _post(scd1, p["dec1"]["b"], 4, 32)

    hd2, asd2, add2 = _pre(rawd1, (sd11, sd21), p["dbn1"], p["dec2"]["W"],
                           *_att(p["dec2"], 1, 32), 1, 32, 128, False)
    scd2 = _sc_gat(src, dst, hd2, asd2, add2, 1, 32)
    rawd2, sd12, sd22 = _post(scd2, p["dec2"]["b"], 1, 32)

    Wd3 = jnp.pad(p["dec3"]["W"], ((0, 0), (0, 5)))
    hd3, asd3, add3, d2 = _pre(rawd2, (sd12, sd22), p["dbn2"], Wd3,
                               *_att(p["dec3"], 1, 11, 16), 1, 16, 32, True)
    scd3 = _sc_gat(src, dst, hd3, asd3, add3, 1, 16)
    bd3 = jnp.pad(p["dec3"]["b"], (0, 5))
    contp, canp = _final(scd3, bd3, d2, p["cls"]["W"], p["cls"]["b"])

    cont = contp[:N, :11]
    canid = canp[:N]
    z = zp[:N]
    kl = -0.5 * klp[0, 0] / (N * LAT)
    return cont, canid, z, kl


# speedup vs baseline: 30.8627x; 1.0006x over previous
import functools
import jax
import jax.numpy as jnp
from jax import lax
from jax.experimental import pallas as pl
from jax.experimental.pallas import tpu as pltpu
from jax.experimental.pallas import tpu_sc as plsc

N = 50000
RB = 128
TN = 50048
GRID = TN // RB
NUM_IDS = 2048
IN_CH = 12
HID = 32
LAT = 32
F32 = jnp.float32
NWORK = 32
ECHUNK = 128


def _sc_gat(src, dst, htab, asT, adT, H, C):
    SUB = TN // 16
    ZC = 136
    NZ = SUB // ZC
    ZD = 3136
    EP = src.shape[0]
    EPW = EP // NWORK
    CHUNKS = EPW // ECHUNK
    mesh = plsc.VectorSubcoreMesh(core_axis_name="c", subcore_axis_name="s")

    @functools.partial(
        pl.kernel,
        out_type=(jax.ShapeDtypeStruct((2, H, TN, C), F32),
                  jax.ShapeDtypeStruct((H * 2 * TN,), F32)),
        mesh=mesh,
        compiler_params=pltpu.CompilerParams(use_tc_tiling_on_sc=False),
        scratch_types=[
            pltpu.VMEM((2, ECHUNK), jnp.int32),
            pltpu.VMEM((2, ECHUNK), jnp.int32),
            pltpu.VMEM((2, ECHUNK), jnp.int32),
            pltpu.VMEM((2, ECHUNK), jnp.int32),
            pltpu.VMEM((2, ECHUNK), F32),
            pltpu.VMEM((2, ECHUNK), F32),
            pltpu.VMEM((2, ECHUNK), F32),
            pltpu.VMEM((2, ECHUNK, C), F32),
            pltpu.VMEM((2, ECHUNK, C), F32),
            pltpu.VMEM((ZC, C), F32),
            pltpu.VMEM((ZD,), F32),
            pltpu.VMEM_SHARED((TN, C), F32),
            pltpu.VMEM_SHARED((TN,), F32),
            pltpu.SemaphoreType.DMA,
            pltpu.SemaphoreType.DMA,
        ],
    )
    def k(src_hbm, dst_hbm, htab_hbm, asT_hbm, adT_hbm, num_hbm, den_hbm,
          src_v, dst_v, idxs_v, idxd_v, as_v, ad_v, w_v, rows_v, val_v,
          zero_v, zd_v, acc_n, acc_d, sem0, sem1):
        cid = lax.axis_index("c")
        sid = lax.axis_index("s")
        wbase = (cid * 16 + sid) * EPW
        zrow = sid * SUB
        z16 = jnp.zeros((16,), F32)

        def zfill(r, carry):
            for c0 in range(0, C, 16):
                zero_v[r, pl.ds(c0, 16)] = z16
            return carry
        lax.fori_loop(0, ZC, zfill, 0)

        def zdfill(g, carry):
            zd_v[pl.ds(g * 16, 16)] = z16
            return carry
        lax.fori_loop(0, ZD // 16, zdfill, 0)

        for h in range(H):
            def zero_body(j, carry):
                pltpu.sync_copy(zero_v, acc_n.at[pl.ds(zrow + j * ZC, ZC)])
                return carry
            lax.fori_loop(0, NZ, zero_body, 0)
            pltpu.sync_copy(zd_v.at[pl.ds(0, SUB)], acc_d.at[pl.ds(zrow, SUB)])
            plsc.subcore_barrier()

            sems = (sem0, sem1)

            def load_and_issue(i, b):
                base = wbase + i * ECHUNK
                pltpu.sync_copy(src_hbm.at[pl.ds(base, ECHUNK)], src_v.at[b])
                pltpu.sync_copy(dst_hbm.at[pl.ds(base, ECHUNK)], dst_v.at[b])
                if h == 0:
                    s_idx, d_idx = src_v.at[b], dst_v.at[b]
                else:
                    def addoff(g, c2):
                        sl = pl.ds(g * 16, 16)
                        idxs_v[b, sl] = src_v[b, sl] + (h * TN)
                        idxd_v[b, sl] = dst_v[b, sl] + (h * TN)
                        return c2
                    lax.fori_loop(0, ECHUNK // 16, addoff, 0)
                    s_idx, d_idx = idxs_v.at[b], idxd_v.at[b]
                pltpu.async_copy(asT_hbm.at[s_idx], as_v.at[b], sems[b])
                pltpu.async_copy(adT_hbm.at[d_idx], ad_v.at[b], sems[b])
                pltpu.async_copy(htab_hbm.at[h].at[src_v.at[b]],
                                 rows_v.at[b], sems[b])

            def wait_gathers(b):
                pltpu.make_async_copy(asT_hbm.at[pl.ds(0, ECHUNK)],
                                      as_v.at[b], sems[b]).wait()
                pltpu.make_async_copy(adT_hbm.at[pl.ds(0, ECHUNK)],
                                      ad_v.at[b], sems[b]).wait()
                pltpu.make_async_copy(htab_hbm.at[h].at[pl.ds(0, ECHUNK)],
                                      rows_v.at[b], sems[b]).wait()

            def compute_scatter(i, b):
                def gbody(g, c2):
                    sl = pl.ds(g * 16, 16)
                    a16 = as_v[b, sl] + ad_v[b, sl]
                    a16 = jnp.where(a16 > 0, a16, a16 * 0.2)
                    w16 = jnp.exp(a16)
                    w_v[b, sl] = w16
                    for j in range(16):
                        e = g * 16 + j
                        wv = w16[j]
                        for c0 in range(0, C, 16):
                            csl = pl.ds(c0, 16)
                            val_v[b, e, csl] = rows_v[b, e, csl] * wv
                    return c2
                lax.fori_loop(0, ECHUNK // 16, gbody, 0)
                pltpu.sync_copy(val_v.at[b], acc_n.at[dst_v.at[b]], add=True)
                pltpu.sync_copy(w_v.at[b], acc_d.at[dst_v.at[b]], add=True)

            load_and_issue(0, 0)

            def dchunk(i2, carry):
                for b in range(2):
                    i = i2 * 2 + b
                    nb = 1 - b

                    @pl.when(i + 1 < CHUNKS)
                    def _():
                        load_and_issue(i + 1, nb)
                    wait_gathers(b)
                    compute_scatter(i, b)
                return carry
            lax.fori_loop(0, CHUNKS // 2, dchunk, 0)
            plsc.subcore_barrier()

            pltpu.sync_copy(acc_n.at[pl.ds(zrow, SUB)],
                            num_hbm.at[cid, h, pl.ds(zrow, SUB)])
            pltpu.sync_copy(
                acc_d.at[pl.ds(zrow, SUB)],
                den_hbm.at[pl.ds(h * 2 * TN + cid * TN + zrow, SUB)])
            plsc.subcore_barrier()

    num, den = k(src, dst, htab, asT.reshape(-1), adT.reshape(-1))
    return num, den.reshape(H, 2, TN)


def _full(shape):
    nd = len(shape)
    return pl.BlockSpec(shape, lambda i: (0,) * nd)


def _pre1(x, emb, W, a_s, a_d):
    def body(x_ref, emb_ref, W_ref, as_ref, ad_ref, ht_ref, asT_ref, adT_ref):
        xb = x_ref[...]
        ids = xb[:, 0].astype(jnp.int32)
        oh = (lax.broadcasted_iota(jnp.int32, (RB, NUM_IDS), 1)
              == ids[:, None]).astype(F32)
        e = jnp.dot(oh, emb_ref[...], preferred_element_type=F32)
        u = jnp.concatenate([e, xb[:, 1:]], axis=1)
        for h in range(4):
            hh = jnp.dot(u, W_ref[:, h * 32:(h + 1) * 32],
                         preferred_element_type=F32)
            ht_ref[h] = hh
            asT_ref[h] = jnp.sum(hh * as_ref[h][None, :], axis=1)
            adT_ref[h] = jnp.sum(hh * ad_ref[h][None, :], axis=1)

    return pl.pallas_call(
        body,
        grid=(GRID,),
        in_specs=[
            pl.BlockSpec((RB, IN_CH), lambda i: (i, 0)),
            _full((NUM_IDS, 8)),
            _full((19, 128)),
            _full((8, 32)),
            _full((8, 32)),
        ],
        out_specs=[
            pl.BlockSpec((4, RB, 32), lambda i: (0, i, 0)),
            pl.BlockSpec((8, RB), lambda i: (0, i)),
            pl.BlockSpec((8, RB), lambda i: (0, i)),
        ],
        out_shape=[
            jax.ShapeDtypeStruct((4, TN, 32), F32),
            jax.ShapeDtypeStruct((8, TN), F32),
            jax.ShapeDtypeStruct((8, TN), F32),
        ],
    )(x, emb, W, a_s, a_d)


def _pre(u, stats, bn, W, a_s, a_d, H, C, Fin, emit_y):
    with_bn = bn is not None

    def body(*refs):
        idx = 0
        u_ref = refs[idx]; idx += 1
        if with_bn:
            s1_ref = refs[idx]; s2_ref = refs[idx + 1]
            g_ref = refs[idx + 2]; be_ref = refs[idx + 3]
            idx += 4
        W_ref = refs[idx]; as_ref = refs[idx + 1]; ad_ref = refs[idx + 2]
        idx += 3
        ht_ref = refs[idx]; asT_ref = refs[idx + 1]; adT_ref = refs[idx + 2]
        idx += 3
        ub = u_ref[...]
        if with_bn:
            m = s1_ref[0] * (1.0 / N)
            v = s2_ref[0] * (1.0 / N) - m * m
            ub = (g_ref[0][None, :] * (ub - m[None, :])
                  * lax.rsqrt(v + 1e-5)[None, :] + be_ref[0][None, :])
            ub = jnp.maximum(ub, 0.0)
        if emit_y:
            refs[idx][...] = ub
        for h in range(H):
            hh = jnp.dot(ub, W_ref[:, h * C:(h + 1) * C],
                         preferred_element_type=F32)
            ht_ref[h] = hh
            asT_ref[h] = jnp.sum(hh * as_ref[h][None, :], axis=1)
            adT_ref[h] = jnp.sum(hh * ad_ref[h][None, :], axis=1)

    in_specs = [pl.BlockSpec((RB, Fin), lambda i: (i, 0))]
    args = [u]
    if with_bn:
        s1, s2 = stats
        in_specs += [_full((1, Fin))] * 4
        args += [s1, s2, bn["g"].reshape(1, Fin), bn["be"].reshape(1, Fin)]
    in_specs += [_full((Fin, H * C)), _full((8, C)), _full((8, C))]
    args += [W, a_s, a_d]
    out_specs = [
        pl.BlockSpec((H, RB, C), lambda i: (0, i, 0)),
        pl.BlockSpec((8, RB), lambda i: (0, i)),
        pl.BlockSpec((8, RB), lambda i: (0, i)),
    ]
    out_shape = [
        jax.ShapeDtypeStruct((H, TN, C), F32),
        jax.ShapeDtypeStruct((8, TN), F32),
        jax.ShapeDtypeStruct((8, TN), F32),
    ]
    if emit_y:
        out_specs.append(pl.BlockSpec((RB, Fin), lambda i: (i, 0)))
        out_shape.append(jax.ShapeDtypeStruct((TN, Fin), F32))
    return pl.pallas_call(
        body, grid=(GRID,), in_specs=in_specs,
        out_specs=out_specs, out_shape=out_shape,
    )(*args)


def _post(sc, b, H, C):
    F = H * C
    scn, scd = sc

    def body(sc_ref, dn_ref, b_ref, raw_ref, s1_ref, s2_ref):
        so = sc_ref[...]
        dn = dn_ref[...]
        num = so[0] + so[1]
        den = dn[:, 0] + dn[:, 1]
        outp = num / (den[:, :, None] + 1e-16)
        raw = jnp.concatenate([outp[h] for h in range(H)], axis=1) + b_ref[0]
        raw_ref[...] = raw
        i = pl.program_id(0)
        mask = (lax.broadcasted_iota(jnp.int32, (RB, 1), 0) + i * RB) < N
        rm = jnp.where(mask, raw, 0.0)
        p1 = jnp.sum(rm, axis=0, keepdims=True)
        p2 = jnp.sum(rm * rm, axis=0, keepdims=True)

        @pl.when(i == 0)
        def _():
            s1_ref[...] = p1
            s2_ref[...] = p2

        @pl.when(i != 0)
        def _():
            s1_ref[...] += p1
            s2_ref[...] += p2

    return pl.pallas_call(
        body,
        grid=(GRID,),
        in_specs=[
            pl.BlockSpec((2, H, RB, C), lambda i: (0, 0, i, 0)),
            pl.BlockSpec((H, 2, RB), lambda i: (0, 0, i)),
            _full((1, F)),
        ],
        out_specs=[
            pl.BlockSpec((RB, F), lambda i: (i, 0)),
            pl.BlockSpec((1, F), lambda i: (0, 0)),
            pl.BlockSpec((1, F), lambda i: (0, 0)),
        ],
        out_shape=[
            jax.ShapeDtypeStruct((TN, F), F32),
            jax.ShapeDtypeStruct((1, F), F32),
            jax.ShapeDtypeStruct((1, F), F32),
        ],
    )(scn, scd, b.reshape(1, F))


def _zk(raw3, s1, s2, bn, y2, zm, zl, eps):
    def body(r_ref, s1_ref, s2_ref, g_ref, be_ref, y2_ref, zmW_ref, zmb_ref,
             zlW_ref, zlb_ref, eps_ref, z_ref, klp_ref):
        m = s1_ref[0] * (1.0 / N)
        v = s2_ref[0] * (1.0 / N) - m * m
        x3 = (g_ref[0][None, :] * (r_ref[...] - m[None, :])
              * lax.rsqrt(v + 1e-5)[None, :] + be_ref[0][None, :])
        x3 = jnp.maximum(x3, 0.0)
        hh = x3 + y2_ref[...]
        mu = jnp.dot(hh, zmW_ref[...], preferred_element_type=F32) + zmb_ref[0]
        lv = jnp.dot(hh, zlW_ref[...], preferred_element_type=F32) + zlb_ref[0]
        z = mu + eps_ref[...] * jnp.exp(0.5 * lv)
        z_ref[...] = z
        i = pl.program_id(0)
        mask = (lax.broadcasted_iota(jnp.int32, (RB, 1), 0) + i * RB) < N
        t = jnp.where(mask, 1.0 + lv - mu * mu - jnp.exp(lv), 0.0)
        p = jnp.sum(t).reshape(1, 1)

        @pl.when(i == 0)
        def _():
            klp_ref[...] = p

        @pl.when(i != 0)
        def _():
            klp_ref[...] += p

    return pl.pallas_call(
        body,
        grid=(GRID,),
        in_specs=[
            pl.BlockSpec((RB, HID), lambda i: (i, 0)),
            _full((1, HID)), _full((1, HID)), _full((1, HID)), _full((1, HID)),
            pl.BlockSpec((RB, HID), lambda i: (i, 0)),
            _full((HID, LAT)), _full((1, LAT)),
            _full((HID, LAT)), _full((1, LAT)),
            pl.BlockSpec((RB, LAT), lambda i: (i, 0)),
        ],
        out_specs=[
            pl.BlockSpec((RB, LAT), lambda i: (i, 0)),
            pl.BlockSpec((1, 1), lambda i: (0, 0)),
        ],
        out_shape=[
            jax.ShapeDtypeStruct((TN, LAT), F32),
            jax.ShapeDtypeStruct((1, 1), F32),
        ],
    )(raw3, s1, s2, bn["g"].reshape(1, HID), bn["be"].reshape(1, HID),
      y2, zm["W"], zm["b"].reshape(1, LAT), zl["W"], zl["b"].reshape(1, LAT),
      eps)


def _final(sc, b, d2, cW, cb):
    scn, scd = sc

    def body(sc_ref, dn_ref, b_ref, d2_ref, cW_ref, cb_ref, cont_ref, can_ref):
        so = sc_ref[...]
        dn = dn_ref[...]
        num = so[0, 0] + so[1, 0]
        den = dn[0, 0] + dn[0, 1]
        raw = num / (den[:, None] + 1e-16) + b_ref[0]
        cont_ref[...] = 1.0 / (1.0 + jnp.exp(-raw))
        can_ref[...] = (jnp.dot(d2_ref[...], cW_ref[...],
                                preferred_element_type=F32) + cb_ref[0])

    return pl.pallas_call(
        body,
        grid=(GRID,),
        in_specs=[
            pl.BlockSpec((2, 1, RB, 16), lambda i: (0, 0, i, 0)),
            pl.BlockSpec((1, 2, RB), lambda i: (0, 0, i)),
            _full((1, 16)),
            pl.BlockSpec((RB, HID), lambda i: (i, 0)),
            _full((HID, NUM_IDS)),
            _full((1, NUM_IDS)),
        ],
        out_specs=[
            pl.BlockSpec((RB, 16), lambda i: (i, 0)),
            pl.BlockSpec((RB, NUM_IDS), lambda i: (i, 0)),
        ],
        out_shape=[
            jax.ShapeDtypeStruct((TN, 16), F32),
            jax.ShapeDtypeStruct((TN, NUM_IDS), F32),
        ],
    )(scn, scd, b.reshape(1, 16), d2, cW, cb.reshape(1, NUM_IDS))


def _att(pr, H, C, Cpad=None):
    Cp = Cpad or C
    a_s = jnp.pad(pr["att_src"].reshape(H, C), ((0, 8 - H), (0, Cp - C)))
    a_d = jnp.pad(pr["att_dst"].reshape(H, C), ((0, 8 - H), (0, Cp - C)))
    return a_s, a_d


def kernel(x, edge_index, batch, params):
    p = params
    xp = jnp.pad(x, ((0, TN - N), (0, 0)))
    loop = jnp.arange(N, dtype=edge_index.dtype)
    src = jnp.concatenate([edge_index[0], loop])
    dst = jnp.concatenate([edge_index[1], loop])
    EE = src.shape[0]
    EQ = NWORK * ECHUNK * 2
    EP = ((EE + EQ - 1) // EQ) * EQ
    src = jnp.pad(src, (0, EP - EE))
    dst = jnp.pad(dst, (0, EP - EE), constant_values=N)
    eps = jax.random.normal(jax.random.key(1), (N, LAT), F32)
    epsp = jnp.pad(eps, ((0, TN - N), (0, 0)))

    h1, as1, ad1 = _pre1(xp, p["emb"], p["enc1"]["W"], *_att(p["enc1"], 4, 32))
    sc1 = _sc_gat(src, dst, h1, as1, ad1, 4, 32)
    raw1, s11, s21 = _post(sc1, p["enc1"]["b"], 4, 32)

    h2, as2, ad2 = _pre(raw1, (s11, s21), p["bn1"], p["enc2"]["W"],
                        *_att(p["enc2"], 1, 32), 1, 32, 128, False)
    sc2 = _sc_gat(src, dst, h2, as2, ad2, 1, 32)
    raw2, s12, s22 = _post(sc2, p["enc2"]["b"], 1, 32)

    h3, as3, ad3, y2 = _pre(raw2, (s12, s22), p["bn2"], p["enc3"]["W"],
                            *_att(p["enc3"], 1, 32), 1, 32, 32, True)
    sc3 = _sc_gat(src, dst, h3, as3, ad3, 1, 32)
    raw3, s13, s23 = _post(sc3, p["enc3"]["b"], 1, 32)

    zp, klp = _zk(raw3, s13, s23, p["bn3"], y2, p["zm"], p["zl"], epsp)

    hd1, asd1, add1 = _pre(zp, None, None, p["dec1"]["W"],
                           *_att(p["dec1"], 4, 32), 4, 32, LAT, False)
    scd1 = _sc_gat(src, dst, hd1, asd1, add1, 4, 32)
    rawd1, sd11, sd21 = _post(scd1, p["dec1"]["b"], 4, 32)

    hd2, asd2, add2 = _pre(rawd1, (sd11, sd21), p["dbn1"], p["dec2"]["W"],
                           *_att(p["dec2"], 1, 32), 1, 32, 128, False)
    scd2 = _sc_gat(src, dst, hd2, asd2, add2, 1, 32)
    rawd2, sd12, sd22 = _post(scd2, p["dec2"]["b"], 1, 32)

    Wd3 = jnp.pad(p["dec3"]["W"], ((0, 0), (0, 5)))
    hd3, asd3, add3, d2 = _pre(rawd2, (sd12, sd22), p["dbn2"], Wd3,
                               *_att(p["dec3"], 1, 11, 16), 1, 16, 32, True)
    scd3 = _sc_gat(src, dst, hd3, asd3, add3, 1, 16)
    bd3 = jnp.pad(p["dec3"]["b"], (0, 5))
    contp, canp = _final(scd3, bd3, d2, p["cls"]["W"], p["cls"]["b"])

    cont = contp[:N, :11]
    canid = canp[:N]
    z = zp[:N]
    kl = -0.5 * klp[0, 0] / (N * LAT)
    return cont, canid, z, kl
